# Initial kernel scaffold; baseline (speedup 1.0000x reference)
import jax
import jax.numpy as jnp
from jax.experimental import pallas as pl


def kernel(x, Wg, W1, W2, W3):
    raise NotImplementedError("write your pallas kernel here")


# speedup vs baseline: 2.1641x; 2.1641x over previous
import functools

import jax
import jax.numpy as jnp
from jax.experimental import pallas as pl
from jax.experimental.pallas import tpu as pltpu

_E = 8
_N_ACTIVE = 4
_TOP_K = 2
_TH = 256

_DN_T = (((1,), (1,)), ((), ()))
_DN_N = (((1,), (0,)), ((), ()))


def _router_body(x_ref, wg_ref, w_ref):
    xb = x_ref[...]
    wg = wg_ref[...]
    x_hi = xb.astype(jnp.bfloat16)
    g_hi = wg.astype(jnp.bfloat16)
    lg = jax.lax.dot_general(g_hi, x_hi, _DN_T, preferred_element_type=jnp.float32)
    tb = lg.shape[1]
    srow = jax.lax.broadcasted_iota(jnp.int32, (_E, tb), 0)
    act = srow < _N_ACTIVE
    neg = jnp.float32(-1e30)
    m = jnp.max(jnp.where(act, lg, neg), axis=0, keepdims=True)
    ex = jnp.where(act, jnp.exp(lg - m), 0.0)
    s = jnp.sum(ex, axis=0, keepdims=True)
    w = ex / s
    m1 = jnp.max(w, axis=0, keepdims=True)
    i1 = jnp.min(jnp.where(w == m1, srow, _E), axis=0, keepdims=True)
    wm = jnp.where(srow == i1, -1.0, w)
    m2 = jnp.max(wm, axis=0, keepdims=True)
    i2 = jnp.min(jnp.where(wm == m2, srow, _E), axis=0, keepdims=True)
    sel = (srow == i1) | (srow == i2)
    w_ref[...] = jnp.where(sel, w / (m1 + m2), 0.0)


def _moe_body(x_ref, w_ref, w1_ref, w3_ref, w2_ref, out_ref):
    e = pl.program_id(0)
    h = pl.program_id(1)
    xb = x_ref[...]
    w1 = w1_ref[0].astype(jnp.bfloat16)
    w3 = w3_ref[0].astype(jnp.bfloat16)
    w2 = w2_ref[0].astype(jnp.bfloat16)
    a = jax.lax.dot_general(w1, xb, _DN_T, preferred_element_type=jnp.float32)
    b = jax.lax.dot_general(w3, xb, _DN_T, preferred_element_type=jnp.float32)
    ht = (a * jax.nn.sigmoid(a)) * b
    srow = jax.lax.broadcasted_iota(jnp.int32, w_ref.shape, 0)
    wrow = jnp.sum(jnp.where(srow == e, w_ref[...], 0.0), axis=0, keepdims=True)
    hw = (ht * wrow).astype(jnp.bfloat16)
    acc = jax.lax.dot_general(w2, hw, _DN_N, preferred_element_type=jnp.float32)

    @pl.when((e == 0) & (h == 0))
    def _init():
        out_ref[...] = acc

    @pl.when((e > 0) | (h > 0))
    def _accum():
        out_ref[...] += acc


@functools.partial(jax.jit, static_argnames=())
def kernel(x, Wg, W1, W2, W3):
    B, S, D = x.shape
    T = B * S
    HID = W1.shape[1]
    xf = x.reshape(T, D)

    w = pl.pallas_call(
        _router_body,
        grid=(4,),
        in_specs=[
            pl.BlockSpec((T // 4, D), lambda i: (i, 0)),
            pl.BlockSpec((_E, D), lambda i: (0, 0)),
        ],
        out_specs=pl.BlockSpec((_E, T // 4), lambda i: (0, i)),
        out_shape=jax.ShapeDtypeStruct((_E, T), jnp.float32),
        compiler_params=pltpu.CompilerParams(
            dimension_semantics=("arbitrary",),
        ),
    )(xf, Wg)

    xb = xf.astype(jnp.bfloat16)
    out_t = pl.pallas_call(
        _moe_body,
        grid=(_N_ACTIVE, HID // _TH),
        in_specs=[
            pl.BlockSpec((T, D), lambda e, h: (0, 0)),
            pl.BlockSpec((_E, T), lambda e, h: (0, 0)),
            pl.BlockSpec((1, _TH, D), lambda e, h: (e, h, 0)),
            pl.BlockSpec((1, _TH, D), lambda e, h: (e, h, 0)),
            pl.BlockSpec((1, D, _TH), lambda e, h: (e, 0, h)),
        ],
        out_specs=pl.BlockSpec((D, T), lambda e, h: (0, 0)),
        out_shape=jax.ShapeDtypeStruct((D, T), jnp.float32),
        compiler_params=pltpu.CompilerParams(
            dimension_semantics=("arbitrary", "arbitrary"),
        ),
    )(xb, w, W1, W3, W2)

    return out_t.T.reshape(B, S, D)


# speedup vs baseline: 2.3625x; 1.0917x over previous
import functools

import jax
import jax.numpy as jnp
from jax import lax
from jax.experimental import pallas as pl
from jax.experimental.pallas import tpu as pltpu, tpu_sc as plsc

_E = 8
_N_ACTIVE = 4
_TH = 256
_TM = 512
_R = 8192 + _N_ACTIVE * _TM
_NT = _R // _TM
_D = 1024
_T = 4096

_DN_T = (((1,), (1,)), ((), ()))
_DN_N = (((1,), (0,)), ((), ()))
_DN_O = (((0,), (0,)), ((), ()))


def _router_body(x_ref, wg_ref, meta_ref):
    xb = x_ref[...]
    wg = wg_ref[...]
    x_hi = xb.astype(jnp.bfloat16)
    g_hi = wg.astype(jnp.bfloat16)
    lg = jax.lax.dot_general(g_hi, x_hi, _DN_T, preferred_element_type=jnp.float32)
    tb = lg.shape[1]
    srow = jax.lax.broadcasted_iota(jnp.int32, (_E, tb), 0)
    act = srow < _N_ACTIVE
    neg = jnp.float32(-1e30)
    m = jnp.max(jnp.where(act, lg, neg), axis=0, keepdims=True)
    ex = jnp.where(act, jnp.exp(lg - m), 0.0)
    s = jnp.sum(ex, axis=0, keepdims=True)
    w = ex / s
    m1 = jnp.max(w, axis=0, keepdims=True)
    i1 = jnp.min(jnp.where(w == m1, srow, _E), axis=0, keepdims=True)
    wm = jnp.where(srow == i1, -1.0, w)
    m2 = jnp.max(wm, axis=0, keepdims=True)
    i2 = jnp.min(jnp.where(wm == m2, srow, _E), axis=0, keepdims=True)
    den = m1 + m2
    meta = jnp.where(srow == 0, i1.astype(jnp.float32),
           jnp.where(srow == 1, i2.astype(jnp.float32),
           jnp.where(srow == 2, m1 / den,
           jnp.where(srow == 3, m2 / den, 0.0))))
    meta_ref[...] = meta


def _book_body(meta_ref, pos_ref, gid_ref, u_s, prefix_s, carry_s, offs_s):
    k = pl.program_id(0)
    tck = meta_ref.shape[1]

    @pl.when(k == 0)
    def _build_u():
        r = jax.lax.broadcasted_iota(jnp.int32, (tck, tck), 0)
        c = jax.lax.broadcasted_iota(jnp.int32, (tck, tck), 1)
        u_s[...] = (r < c).astype(jnp.bfloat16)

    g = meta_ref[...]
    i1r = g[0:1]
    i2r = g[1:2]
    srowf = jax.lax.broadcasted_iota(jnp.int32, (_E, tck), 0).astype(jnp.float32)
    oh1 = srowf == i1r
    oh2 = srowf == i2r
    cnt = oh1.astype(jnp.float32) + oh2.astype(jnp.float32)
    cv = jnp.where(k == 0, 0.0, carry_s[...])

    @pl.when(k < 4)
    def _prefix():
        pref_loc = jax.lax.dot_general(cnt.astype(jnp.bfloat16), u_s[...],
                                       _DN_N, preferred_element_type=jnp.float32)
        prefix_s[pl.ds(8 * (k % 4), 8), :] = pref_loc + cv[:, 0:1]
        carry_s[...] = cv + jnp.sum(cnt, axis=1, keepdims=True)

    @pl.when(k == 4)
    def _offsets():
        c = cv[:, 0:1]
        al = jnp.ceil(c / _TM) * _TM
        io0 = jax.lax.broadcasted_iota(jnp.int32, (8, 1), 0)
        offs = jnp.zeros((8, 1), jnp.float32)
        jtf = jax.lax.broadcasted_iota(jnp.int32, (8, 128), 1).astype(jnp.float32)
        fill = jnp.zeros((8, 128), jnp.float32)
        valid = jnp.zeros((8, 128), jnp.bool_)
        for e in range(_N_ACTIVE):
            ae = jnp.sum(jnp.where(io0 == e, al, 0.0), axis=0, keepdims=True)
            offs = offs + jnp.where(io0 > e, ae, 0.0)
        for e in range(_N_ACTIVE):
            oe = jnp.sum(jnp.where(io0 == e, offs, 0.0), axis=0, keepdims=True)
            ae = jnp.sum(jnp.where(io0 == e, al, 0.0), axis=0, keepdims=True)
            se = oe / _TM
            ne = ae / _TM
            fill = fill + (jtf >= se).astype(jnp.float32)
            valid = valid | ((jtf >= se) & (jtf < se + ne))
        offs_s[...] = offs + jnp.zeros((8, 128), jnp.float32)
        io = jax.lax.broadcasted_iota(jnp.int32, (8, 128), 0)
        gidf = jnp.maximum(fill - 1.0, 0.0)
        gid_ref[...] = jnp.where(io == 0, gidf,
                       jnp.where(io == 1, valid.astype(jnp.float32),
                                 0.0)).astype(jnp.int32)

    @pl.when(k >= 4)
    def _positions():
        pref = prefix_s[pl.ds(8 * (k % 4), 8), :]
        posmat = pref + offs_s[...][:, 0:1]
        p1 = jnp.sum(jnp.where(oh1, posmat, 0.0), axis=0, keepdims=True)
        p2 = jnp.sum(jnp.where(oh2, posmat, 0.0), axis=0, keepdims=True)
        pos_ref[...] = jnp.where(srowf == 0.0, p1,
                       jnp.where(srowf == 1.0, p2, 0.0))


def _gmm_body(gid_ref, valid_ref, xs_ref, w1a_ref, w1b_ref, w3a_ref, w3b_ref,
              w2a_ref, w2b_ref, ys_ref):
    h = pl.program_id(1)
    wrefs = ((w1a_ref, w3a_ref, w2a_ref), (w1b_ref, w3b_ref, w2b_ref))
    for kk in range(2):
        w1r, w3r, w2r = wrefs[kk]
        xsb = xs_ref[pl.ds(kk * _TM, _TM), :].astype(jnp.bfloat16)
        w1 = w1r[0].astype(jnp.bfloat16)
        w3 = w3r[0].astype(jnp.bfloat16)
        w2 = w2r[0].astype(jnp.bfloat16)
        a = jax.lax.dot_general(xsb, w1, _DN_T, preferred_element_type=jnp.float32)
        b = jax.lax.dot_general(xsb, w3, _DN_T, preferred_element_type=jnp.float32)
        ht = (a * jax.nn.sigmoid(a)) * b
        hw = ht.astype(jnp.bfloat16)
        part = jax.lax.dot_general(hw, w2, _DN_T, preferred_element_type=jnp.float32)

        @pl.when(h == 0)
        def _init():
            ys_ref[pl.ds(kk * _TM, _TM), :] = part

        @pl.when(h > 0)
        def _accum():
            ys_ref[pl.ds(kk * _TM, _TM), :] += part


def _scale_body(y1_ref, y2_ref, meta_ref, out_ref):
    ones = jnp.ones((1, _D), jnp.bfloat16)

    def outer(v):
        vh = v.astype(jnp.bfloat16)
        vl = (v - vh.astype(jnp.float32)).astype(jnp.bfloat16)
        return (jax.lax.dot_general(vh, ones, _DN_O, preferred_element_type=jnp.float32)
                + jax.lax.dot_general(vl, ones, _DN_O, preferred_element_type=jnp.float32))

    vb1 = outer(meta_ref[2:3, :])
    vb2 = outer(meta_ref[3:4, :])
    out_ref[...] = vb1 * y1_ref[...] + vb2 * y2_ref[...]


_SC_MESH = plsc.VectorSubcoreMesh(core_axis_name="c", subcore_axis_name="s")
_NW = 32
_SUB = 64


@functools.partial(
    pl.kernel, mesh=_SC_MESH,
    out_type=jax.ShapeDtypeStruct((_R, _D), jnp.float32),
    scratch_types=[
        pltpu.VMEM((_SUB,), jnp.int32),
        pltpu.VMEM((_SUB,), jnp.int32),
        pltpu.VMEM((_SUB, _D), jnp.float32),
        pltpu.SemaphoreType.DMA,
        pltpu.SemaphoreType.DMA,
    ],
)
def _dispatch(xf_hbm, p1_hbm, p2_hbm, xs_hbm, i1_v, i2_v, rows_v, sem0, sem1):
    wid = lax.axis_index("s") * 2 + lax.axis_index("c")
    for j in range(2):
        r = 2 * wid + j
        pltpu.sync_copy(p1_hbm.at[r], i1_v)
        pltpu.sync_copy(p2_hbm.at[r], i2_v)
        pltpu.sync_copy(xf_hbm.at[pl.ds(r * _SUB, _SUB)], rows_v)
        c0 = pltpu.async_copy(rows_v, xs_hbm.at[i1_v], sem0)
        c1 = pltpu.async_copy(rows_v, xs_hbm.at[i2_v], sem1)
        c0.wait()
        c1.wait()


@functools.partial(
    pl.kernel, mesh=_SC_MESH,
    out_type=[jax.ShapeDtypeStruct((_T, _D), jnp.float32),
              jax.ShapeDtypeStruct((_T, _D), jnp.float32)],
    scratch_types=[
        pltpu.VMEM((_SUB,), jnp.int32),
        pltpu.VMEM((_SUB, _D), jnp.float32),
        pltpu.SemaphoreType.DMA,
    ],
)
def _combine(ys_hbm, p1_hbm, p2_hbm, y1_hbm, y2_hbm, i_v, rows_v, sem):
    wid = lax.axis_index("s") * 2 + lax.axis_index("c")
    for j in range(2):
        r = 2 * wid + j
        for phbm, ohbm in ((p1_hbm, y1_hbm), (p2_hbm, y2_hbm)):
            pltpu.sync_copy(phbm.at[r], i_v)
            pltpu.async_copy(ys_hbm.at[i_v], rows_v, sem).wait()
            pltpu.sync_copy(rows_v, ohbm.at[pl.ds(r * _SUB, _SUB)])


@jax.jit
def kernel(x, Wg, W1, W2, W3):
    B, S, D = x.shape
    T = B * S
    HID = W1.shape[1]
    xf = x.reshape(T, D)

    meta = pl.pallas_call(
        _router_body,
        grid=(4,),
        in_specs=[
            pl.BlockSpec((T // 4, D), lambda i: (i, 0)),
            pl.BlockSpec((_E, D), lambda i: (0, 0)),
        ],
        out_specs=pl.BlockSpec((_E, T // 4), lambda i: (0, i)),
        out_shape=jax.ShapeDtypeStruct((_E, T), jnp.float32),
        compiler_params=pltpu.CompilerParams(
            dimension_semantics=("arbitrary",),
        ),
    )(xf, Wg)

    pos_meta, gidv = pl.pallas_call(
        _book_body,
        grid=(8,),
        in_specs=[pl.BlockSpec((_E, T // 4), lambda k: (0, k % 4))],
        out_specs=[
            pl.BlockSpec((_E, T // 4), lambda k: (0, k % 4)),
            pl.BlockSpec((_E, 128), lambda k: (0, 0)),
        ],
        out_shape=[
            jax.ShapeDtypeStruct((_E, T), jnp.float32),
            jax.ShapeDtypeStruct((_E, 128), jnp.int32),
        ],
        scratch_shapes=[
            pltpu.VMEM((T // 4, T // 4), jnp.bfloat16),
            pltpu.VMEM((32, T // 4), jnp.float32),
            pltpu.VMEM((8, 128), jnp.float32),
            pltpu.VMEM((8, 128), jnp.float32),
        ],
        compiler_params=pltpu.CompilerParams(
            dimension_semantics=("arbitrary",),
        ),
    )(meta)

    pos1 = pos_meta[0].astype(jnp.int32).reshape(T // _SUB, _SUB)
    pos2 = pos_meta[1].astype(jnp.int32).reshape(T // _SUB, _SUB)
    gid = gidv[0, :_NT]
    valid = gidv[1, :_NT]

    xs = _dispatch(xf, pos1, pos2)

    ys = pl.pallas_call(
        _gmm_body,
        grid_spec=pltpu.PrefetchScalarGridSpec(
            num_scalar_prefetch=2,
            grid=(_NT // 2, HID // _TH),
            in_specs=[
                pl.BlockSpec((2 * _TM, D), lambda p, h, g, v: (p, 0)),
                pl.BlockSpec((1, _TH, D), lambda p, h, g, v: (g[2 * p], h, 0)),
                pl.BlockSpec((1, _TH, D), lambda p, h, g, v: (g[2 * p + 1], h, 0)),
                pl.BlockSpec((1, _TH, D), lambda p, h, g, v: (g[2 * p], h, 0)),
                pl.BlockSpec((1, _TH, D), lambda p, h, g, v: (g[2 * p + 1], h, 0)),
                pl.BlockSpec((1, D, _TH), lambda p, h, g, v: (g[2 * p], 0, h)),
                pl.BlockSpec((1, D, _TH), lambda p, h, g, v: (g[2 * p + 1], 0, h)),
            ],
            out_specs=pl.BlockSpec((2 * _TM, D), lambda p, h, g, v: (p, 0)),
        ),
        out_shape=jax.ShapeDtypeStruct((_R, D), jnp.float32),
        compiler_params=pltpu.CompilerParams(
            dimension_semantics=("arbitrary", "arbitrary"),
        ),
    )(gid, valid, xs, W1, W1, W3, W3, W2, W2)

    y1, y2 = _combine(ys, pos1, pos2)

    out = pl.pallas_call(
        _scale_body,
        grid=(8,),
        in_specs=[
            pl.BlockSpec((T // 8, D), lambda i: (i, 0)),
            pl.BlockSpec((T // 8, D), lambda i: (i, 0)),
            pl.BlockSpec((_E, T // 8), lambda i: (0, i)),
        ],
        out_specs=pl.BlockSpec((T // 8, D), lambda i: (i, 0)),
        out_shape=jax.ShapeDtypeStruct((T, D), jnp.float32),
        compiler_params=pltpu.CompilerParams(
            dimension_semantics=("arbitrary",),
        ),
    )(y1, y2, meta)

    return out.reshape(B, S, D)


# speedup vs baseline: 2.4019x; 1.0167x over previous
import functools

import jax
import jax.numpy as jnp
from jax import lax
from jax.experimental import pallas as pl
from jax.experimental.pallas import tpu as pltpu, tpu_sc as plsc

_E = 8
_N_ACTIVE = 4
_TH = 256
_TM = 1024
_R = 8192 + _N_ACTIVE * _TM
_NT = _R // _TM
_D = 1024
_T = 4096

_DN_T = (((1,), (1,)), ((), ()))
_DN_N = (((1,), (0,)), ((), ()))
_DN_O = (((0,), (0,)), ((), ()))


def _router_body(x_ref, wg_ref, meta_ref):
    xb = x_ref[...]
    wg = wg_ref[...]
    x_hi = xb.astype(jnp.bfloat16)
    g_hi = wg.astype(jnp.bfloat16)
    lg = jax.lax.dot_general(g_hi, x_hi, _DN_T, preferred_element_type=jnp.float32)
    tb = lg.shape[1]
    srow = jax.lax.broadcasted_iota(jnp.int32, (_E, tb), 0)
    act = srow < _N_ACTIVE
    neg = jnp.float32(-1e30)
    m = jnp.max(jnp.where(act, lg, neg), axis=0, keepdims=True)
    ex = jnp.where(act, jnp.exp(lg - m), 0.0)
    s = jnp.sum(ex, axis=0, keepdims=True)
    w = ex / s
    m1 = jnp.max(w, axis=0, keepdims=True)
    i1 = jnp.min(jnp.where(w == m1, srow, _E), axis=0, keepdims=True)
    wm = jnp.where(srow == i1, -1.0, w)
    m2 = jnp.max(wm, axis=0, keepdims=True)
    i2 = jnp.min(jnp.where(wm == m2, srow, _E), axis=0, keepdims=True)
    den = m1 + m2
    meta = jnp.where(srow == 0, i1.astype(jnp.float32),
           jnp.where(srow == 1, i2.astype(jnp.float32),
           jnp.where(srow == 2, m1 / den,
           jnp.where(srow == 3, m2 / den, 0.0))))
    meta_ref[...] = meta


def _book_body(meta_ref, pos_ref, gid_ref, u_s, prefix_s, carry_s, offs_s):
    k = pl.program_id(0)
    tck = meta_ref.shape[1]

    @pl.when(k == 0)
    def _build_u():
        r = jax.lax.broadcasted_iota(jnp.int32, (tck, tck), 0)
        c = jax.lax.broadcasted_iota(jnp.int32, (tck, tck), 1)
        u_s[...] = (r < c).astype(jnp.bfloat16)

    g = meta_ref[...]
    i1r = g[0:1]
    i2r = g[1:2]
    srowf = jax.lax.broadcasted_iota(jnp.int32, (_E, tck), 0).astype(jnp.float32)
    oh1 = srowf == i1r
    oh2 = srowf == i2r
    cnt = oh1.astype(jnp.float32) + oh2.astype(jnp.float32)
    cv = jnp.where(k == 0, 0.0, carry_s[...])

    @pl.when(k < 4)
    def _prefix():
        pref_loc = jax.lax.dot_general(cnt.astype(jnp.bfloat16), u_s[...],
                                       _DN_N, preferred_element_type=jnp.float32)
        prefix_s[pl.ds(8 * (k % 4), 8), :] = pref_loc + cv[:, 0:1]
        carry_s[...] = cv + jnp.sum(cnt, axis=1, keepdims=True)

    @pl.when(k == 4)
    def _offsets():
        c = cv[:, 0:1]
        al = jnp.ceil(c / _TM) * _TM
        io0 = jax.lax.broadcasted_iota(jnp.int32, (8, 1), 0)
        offs = jnp.zeros((8, 1), jnp.float32)
        jtf = jax.lax.broadcasted_iota(jnp.int32, (8, 128), 1).astype(jnp.float32)
        fill = jnp.zeros((8, 128), jnp.float32)
        valid = jnp.zeros((8, 128), jnp.bool_)
        for e in range(_N_ACTIVE):
            ae = jnp.sum(jnp.where(io0 == e, al, 0.0), axis=0, keepdims=True)
            offs = offs + jnp.where(io0 > e, ae, 0.0)
        for e in range(_N_ACTIVE):
            oe = jnp.sum(jnp.where(io0 == e, offs, 0.0), axis=0, keepdims=True)
            ae = jnp.sum(jnp.where(io0 == e, al, 0.0), axis=0, keepdims=True)
            se = oe / _TM
            ne = ae / _TM
            fill = fill + (jtf >= se).astype(jnp.float32)
            valid = valid | ((jtf >= se) & (jtf < se + ne))
        offs_s[...] = offs + jnp.zeros((8, 128), jnp.float32)
        io = jax.lax.broadcasted_iota(jnp.int32, (8, 128), 0)
        gidf = jnp.maximum(fill - 1.0, 0.0)
        gid_ref[...] = jnp.where(io == 0, gidf,
                       jnp.where(io == 1, valid.astype(jnp.float32),
                                 0.0)).astype(jnp.int32)

    @pl.when(k >= 4)
    def _positions():
        pref = prefix_s[pl.ds(8 * (k % 4), 8), :]
        posmat = pref + offs_s[...][:, 0:1]
        p1 = jnp.sum(jnp.where(oh1, posmat, 0.0), axis=0, keepdims=True)
        p2 = jnp.sum(jnp.where(oh2, posmat, 0.0), axis=0, keepdims=True)
        pos_ref[...] = jnp.where(srowf == 0.0, p1,
                       jnp.where(srowf == 1.0, p2, 0.0))


def _gmm_body(gid_ref, valid_ref, xs_ref, w1a_ref, w1b_ref, w3a_ref, w3b_ref,
              w2a_ref, w2b_ref, ys_ref):
    h = pl.program_id(1)
    wrefs = ((w1a_ref, w3a_ref, w2a_ref), (w1b_ref, w3b_ref, w2b_ref))
    for kk in range(2):
        w1r, w3r, w2r = wrefs[kk]
        xsb = xs_ref[pl.ds(kk * _TM, _TM), :].astype(jnp.bfloat16)
        w1 = w1r[0].astype(jnp.bfloat16)
        w3 = w3r[0].astype(jnp.bfloat16)
        w2 = w2r[0].astype(jnp.bfloat16)
        a = jax.lax.dot_general(xsb, w1, _DN_T, preferred_element_type=jnp.float32)
        b = jax.lax.dot_general(xsb, w3, _DN_T, preferred_element_type=jnp.float32)
        ht = (a * jax.nn.sigmoid(a)) * b
        hw = ht.astype(jnp.bfloat16)
        part = jax.lax.dot_general(hw, w2, _DN_T, preferred_element_type=jnp.float32)

        @pl.when(h == 0)
        def _init():
            ys_ref[pl.ds(kk * _TM, _TM), :] = part

        @pl.when(h > 0)
        def _accum():
            ys_ref[pl.ds(kk * _TM, _TM), :] += part


def _scale_body(y1_ref, y2_ref, meta_ref, out_ref):
    ones = jnp.ones((1, _D), jnp.bfloat16)

    def outer(v):
        vh = v.astype(jnp.bfloat16)
        vl = (v - vh.astype(jnp.float32)).astype(jnp.bfloat16)
        return (jax.lax.dot_general(vh, ones, _DN_O, preferred_element_type=jnp.float32)
                + jax.lax.dot_general(vl, ones, _DN_O, preferred_element_type=jnp.float32))

    vb1 = outer(meta_ref[2:3, :])
    vb2 = outer(meta_ref[3:4, :])
    out_ref[...] = vb1 * y1_ref[...] + vb2 * y2_ref[...]


_SC_MESH = plsc.VectorSubcoreMesh(core_axis_name="c", subcore_axis_name="s")
_NW = 32
_SUB = 64


@functools.partial(
    pl.kernel, mesh=_SC_MESH,
    out_type=jax.ShapeDtypeStruct((_R, _D), jnp.float32),
    scratch_types=[
        pltpu.VMEM((_SUB,), jnp.int32),
        pltpu.VMEM((_SUB,), jnp.int32),
        pltpu.VMEM((_SUB, _D), jnp.float32),
        pltpu.SemaphoreType.DMA,
        pltpu.SemaphoreType.DMA,
    ],
)
def _dispatch(xf_hbm, p1_hbm, p2_hbm, xs_hbm, i1_v, i2_v, rows_v, sem0, sem1):
    wid = lax.axis_index("s") * 2 + lax.axis_index("c")
    for j in range(2):
        r = 2 * wid + j
        pltpu.sync_copy(p1_hbm.at[r], i1_v)
        pltpu.sync_copy(p2_hbm.at[r], i2_v)
        pltpu.sync_copy(xf_hbm.at[pl.ds(r * _SUB, _SUB)], rows_v)
        c0 = pltpu.async_copy(rows_v, xs_hbm.at[i1_v], sem0)
        c1 = pltpu.async_copy(rows_v, xs_hbm.at[i2_v], sem1)
        c0.wait()
        c1.wait()


@functools.partial(
    pl.kernel, mesh=_SC_MESH,
    out_type=[jax.ShapeDtypeStruct((_T, _D), jnp.float32),
              jax.ShapeDtypeStruct((_T, _D), jnp.float32)],
    scratch_types=[
        pltpu.VMEM((_SUB,), jnp.int32),
        pltpu.VMEM((_SUB, _D), jnp.float32),
        pltpu.SemaphoreType.DMA,
    ],
)
def _combine(ys_hbm, p1_hbm, p2_hbm, y1_hbm, y2_hbm, i_v, rows_v, sem):
    wid = lax.axis_index("s") * 2 + lax.axis_index("c")
    for j in range(2):
        r = 2 * wid + j
        for phbm, ohbm in ((p1_hbm, y1_hbm), (p2_hbm, y2_hbm)):
            pltpu.sync_copy(phbm.at[r], i_v)
            pltpu.async_copy(ys_hbm.at[i_v], rows_v, sem).wait()
            pltpu.sync_copy(rows_v, ohbm.at[pl.ds(r * _SUB, _SUB)])


@jax.jit
def kernel(x, Wg, W1, W2, W3):
    B, S, D = x.shape
    T = B * S
    HID = W1.shape[1]
    xf = x.reshape(T, D)

    meta = pl.pallas_call(
        _router_body,
        grid=(4,),
        in_specs=[
            pl.BlockSpec((T // 4, D), lambda i: (i, 0)),
            pl.BlockSpec((_E, D), lambda i: (0, 0)),
        ],
        out_specs=pl.BlockSpec((_E, T // 4), lambda i: (0, i)),
        out_shape=jax.ShapeDtypeStruct((_E, T), jnp.float32),
        compiler_params=pltpu.CompilerParams(
            dimension_semantics=("arbitrary",),
        ),
    )(xf, Wg)

    pos_meta, gidv = pl.pallas_call(
        _book_body,
        grid=(8,),
        in_specs=[pl.BlockSpec((_E, T // 4), lambda k: (0, k % 4))],
        out_specs=[
            pl.BlockSpec((_E, T // 4), lambda k: (0, k % 4)),
            pl.BlockSpec((_E, 128), lambda k: (0, 0)),
        ],
        out_shape=[
            jax.ShapeDtypeStruct((_E, T), jnp.float32),
            jax.ShapeDtypeStruct((_E, 128), jnp.int32),
        ],
        scratch_shapes=[
            pltpu.VMEM((T // 4, T // 4), jnp.bfloat16),
            pltpu.VMEM((32, T // 4), jnp.float32),
            pltpu.VMEM((8, 128), jnp.float32),
            pltpu.VMEM((8, 128), jnp.float32),
        ],
        compiler_params=pltpu.CompilerParams(
            dimension_semantics=("arbitrary",),
        ),
    )(meta)

    pos1 = pos_meta[0].astype(jnp.int32).reshape(T // _SUB, _SUB)
    pos2 = pos_meta[1].astype(jnp.int32).reshape(T // _SUB, _SUB)
    gid = gidv[0, :_NT]
    valid = gidv[1, :_NT]

    xs = _dispatch(xf, pos1, pos2)

    ys = pl.pallas_call(
        _gmm_body,
        grid_spec=pltpu.PrefetchScalarGridSpec(
            num_scalar_prefetch=2,
            grid=(_NT // 2, HID // _TH),
            in_specs=[
                pl.BlockSpec((2 * _TM, D), lambda p, h, g, v: (p, 0)),
                pl.BlockSpec((1, _TH, D), lambda p, h, g, v: (g[2 * p], h, 0)),
                pl.BlockSpec((1, _TH, D), lambda p, h, g, v: (g[2 * p + 1], h, 0)),
                pl.BlockSpec((1, _TH, D), lambda p, h, g, v: (g[2 * p], h, 0)),
                pl.BlockSpec((1, _TH, D), lambda p, h, g, v: (g[2 * p + 1], h, 0)),
                pl.BlockSpec((1, D, _TH), lambda p, h, g, v: (g[2 * p], 0, h)),
                pl.BlockSpec((1, D, _TH), lambda p, h, g, v: (g[2 * p + 1], 0, h)),
            ],
            out_specs=pl.BlockSpec((2 * _TM, D), lambda p, h, g, v: (p, 0)),
        ),
        out_shape=jax.ShapeDtypeStruct((_R, D), jnp.float32),
        compiler_params=pltpu.CompilerParams(
            dimension_semantics=("arbitrary", "arbitrary"),
        ),
    )(gid, valid, xs, W1, W1, W3, W3, W2, W2)

    y1, y2 = _combine(ys, pos1, pos2)

    out = pl.pallas_call(
        _scale_body,
        grid=(8,),
        in_specs=[
            pl.BlockSpec((T // 8, D), lambda i: (i, 0)),
            pl.BlockSpec((T // 8, D), lambda i: (i, 0)),
            pl.BlockSpec((_E, T // 8), lambda i: (0, i)),
        ],
        out_specs=pl.BlockSpec((T // 8, D), lambda i: (i, 0)),
        out_shape=jax.ShapeDtypeStruct((T, D), jnp.float32),
        compiler_params=pltpu.CompilerParams(
            dimension_semantics=("arbitrary",),
        ),
    )(y1, y2, meta)

    return out.reshape(B, S, D)
